# Initial kernel scaffold; baseline (speedup 1.0000x reference)
import jax
import jax.numpy as jnp
from jax.experimental import pallas as pl
from jax.experimental.pallas import tpu as pltpu

_N_GRAPHS = 512
_BLK = 2500


def _seg_mlp_kernel(idx_ref, nodes_ref, glob_ref, w_ref, b_ref, out_ref, acc_ref):
    i = pl.program_id(0)
    n_blocks = pl.num_programs(0)

    @pl.when(i == 0)
    def _():
        acc_ref[...] = jnp.zeros_like(acc_ref)

    idx = idx_ref[0, 0, :]
    iota = jax.lax.broadcasted_iota(jnp.int32, (_N_GRAPHS, _BLK), 0)
    onehot = (iota == idx[None, :]).astype(jnp.bfloat16)
    nodes_bf = nodes_ref[...].astype(jnp.bfloat16)
    acc_ref[...] += jnp.dot(onehot, nodes_bf, preferred_element_type=jnp.float32)

    @pl.when(i == n_blocks - 1)
    def _():
        d_feat = acc_ref.shape[1]
        w_top = w_ref[:d_feat, :]
        w_bot = w_ref[d_feat:, :]
        out_ref[...] = (
            jnp.dot(acc_ref[...], w_top, preferred_element_type=jnp.float32)
            + jnp.dot(glob_ref[...], w_bot, preferred_element_type=jnp.float32)
            + b_ref[...][None, :]
        )


@jax.jit
def kernel(nodes, edges, receivers, senders, global_latent, node_graph_idx,
           edge_graph_idx, W, b):
    n_nodes, d_feat = nodes.shape
    n_graphs, d_global = global_latent.shape
    d_out = W.shape[1]
    assert n_nodes % _BLK == 0
    n_blocks = n_nodes // _BLK
    idx3 = node_graph_idx.reshape(n_blocks, 1, _BLK)

    return pl.pallas_call(
        _seg_mlp_kernel,
        grid=(n_blocks,),
        in_specs=[
            pl.BlockSpec((1, 1, _BLK), lambda i: (i, 0, 0)),
            pl.BlockSpec((_BLK, d_feat), lambda i: (i, 0)),
            pl.BlockSpec((n_graphs, d_global), lambda i: (0, 0)),
            pl.BlockSpec((d_feat + d_global, d_out), lambda i: (0, 0)),
            pl.BlockSpec((d_out,), lambda i: (0,)),
        ],
        out_specs=pl.BlockSpec((n_graphs, d_out), lambda i: (0, 0)),
        scratch_shapes=[pltpu.VMEM((n_graphs, d_feat), jnp.float32)],
        out_shape=jax.ShapeDtypeStruct((n_graphs, d_out), jnp.float32),
    )(idx3, nodes, global_latent, W, b)


# speedup vs baseline: 6.6596x; 6.6596x over previous
import jax
import jax.numpy as jnp
from jax.experimental import pallas as pl
from jax.experimental.pallas import tpu as pltpu

_N_GRAPHS = 512
_BLK = 2000


def _seg_mlp_kernel(idx_ref, nodes_ref, glob_ref, w_ref, b_ref, out_ref, acc_ref):
    i = pl.program_id(0)
    n_blocks = pl.num_programs(0)

    @pl.when(i == 0)
    def _():
        acc_ref[...] = jnp.zeros_like(acc_ref)

    idx = idx_ref[0, 0, :]
    iota = jax.lax.broadcasted_iota(jnp.int32, (_N_GRAPHS, _BLK), 0)
    onehot = (iota == idx[None, :]).astype(jnp.bfloat16)
    nodes_bf = nodes_ref[...].astype(jnp.bfloat16)
    acc_ref[...] += jnp.dot(onehot, nodes_bf, preferred_element_type=jnp.float32)

    @pl.when(i == n_blocks - 1)
    def _():
        d_feat = acc_ref.shape[1]
        w_top = w_ref[:d_feat, :]
        w_bot = w_ref[d_feat:, :]
        out_ref[...] = (
            jnp.dot(acc_ref[...], w_top, preferred_element_type=jnp.float32)
            + jnp.dot(glob_ref[...], w_bot, preferred_element_type=jnp.float32)
            + b_ref[...][None, :]
        )


@jax.jit
def kernel(nodes, edges, receivers, senders, global_latent, node_graph_idx,
           edge_graph_idx, W, b):
    n_nodes, d_feat = nodes.shape
    n_graphs, d_global = global_latent.shape
    d_out = W.shape[1]
    assert n_nodes % _BLK == 0
    n_blocks = n_nodes // _BLK
    idx3 = node_graph_idx.reshape(n_blocks, 1, _BLK)

    return pl.pallas_call(
        _seg_mlp_kernel,
        grid=(n_blocks,),
        in_specs=[
            pl.BlockSpec((1, 1, _BLK), lambda i: (i, 0, 0)),
            pl.BlockSpec((_BLK, d_feat), lambda i: (i, 0)),
            pl.BlockSpec((n_graphs, d_global), lambda i: (0, 0)),
            pl.BlockSpec((d_feat + d_global, d_out), lambda i: (0, 0)),
            pl.BlockSpec((d_out,), lambda i: (0,)),
        ],
        out_specs=pl.BlockSpec((n_graphs, d_out), lambda i: (0, 0)),
        scratch_shapes=[pltpu.VMEM((n_graphs, d_feat), jnp.float32)],
        out_shape=jax.ShapeDtypeStruct((n_graphs, d_out), jnp.float32),
    )(idx3, nodes, global_latent, W, b)
